# baseline (speedup 1.0000x reference)
import functools

import jax
import jax.numpy as jnp
from jax import lax
from jax.experimental import pallas as pl
from jax.experimental.pallas import tpu as pltpu
from jax.experimental.pallas import tpu_sc as plsc

_NE = 1024
_D = 64
_N = 16384
_CC = 0.25
_BLK = 1024

_NC = 2
_NS = 16
_NW = _NC * _NS
_BPW = _N // _NW


def _tc_body(x_ref, e_ref, enc_ref, idx_ref, loss_ref, ppl_ref,
             counts_ref, sse_ref):
    i = pl.program_id(0)
    x = x_ref[...]
    e = e_ref[...]

    e2 = jnp.sum(e * e, axis=1)
    s = jax.lax.dot_general(x, e, (((1,), (1,)), ((), ())),
                            preferred_element_type=jnp.float32)
    score = 2.0 * s - e2[None, :]
    idx = jnp.argmax(score, axis=1)
    idx_ref[...] = idx[None, None, :]
    onehot = (jax.lax.broadcasted_iota(jnp.int32, (_BLK, _NE), 1)
              == idx[:, None]).astype(jnp.float32)
    enc_ref[...] = onehot

    @pl.when(i == 0)
    def _init():
        counts_ref[...] = jnp.zeros_like(counts_ref)
        sse_ref[0] = 0.0

    counts_ref[...] += jnp.sum(onehot, axis=0, keepdims=True)
    sse_ref[0] += jnp.sum(x * x) - jnp.sum(onehot * score)

    loss_ref[...] = jnp.zeros((1, 1), jnp.float32)
    ppl_ref[...] = jnp.zeros((1, 1), jnp.float32)

    @pl.when(i == pl.num_programs(0) - 1)
    def _final():
        loss_ref[...] = jnp.full((1, 1), _CC * sse_ref[0] / (_N * _D))
        p = counts_ref[...] / _N
        ppl_ref[...] = jnp.exp(-jnp.sum(p * jnp.log(p + 1e-10),
                                        keepdims=True))


def _tc_select(inputs, embedding_weight):
    return pl.pallas_call(
        _tc_body,
        grid=(_N // _BLK,),
        in_specs=[
            pl.BlockSpec((_BLK, _D), lambda i: (i, 0)),
            pl.BlockSpec((_NE, _D), lambda i: (0, 0)),
        ],
        out_specs=[
            pl.BlockSpec((_BLK, _NE), lambda i: (i, 0)),
            pl.BlockSpec((1, 1, _BLK), lambda i: (i, 0, 0)),
            pl.BlockSpec((1, 1), lambda i: (0, 0)),
            pl.BlockSpec((1, 1), lambda i: (0, 0)),
        ],
        out_shape=[
            jax.ShapeDtypeStruct((_N, _NE), jnp.float32),
            jax.ShapeDtypeStruct((_N // _BLK, 1, _BLK), jnp.int32),
            jax.ShapeDtypeStruct((1, 1), jnp.float32),
            jax.ShapeDtypeStruct((1, 1), jnp.float32),
        ],
        scratch_shapes=[
            pltpu.VMEM((1, _NE), jnp.float32),
            pltpu.SMEM((1,), jnp.float32),
        ],
    )(inputs, embedding_weight)


@functools.partial(
    pl.kernel,
    mesh=plsc.VectorSubcoreMesh(core_axis_name="c", subcore_axis_name="s"),
    out_type=jax.ShapeDtypeStruct((_N // 2, 2 * _D), jnp.float32),
    scratch_types=[
        pltpu.VMEM((_BPW,), jnp.int32),
        pltpu.VMEM((_BPW, 128), jnp.float32),
        pltpu.VMEM((_BPW // 2, 128), jnp.float32),
        pltpu.SemaphoreType.DMA,
    ],
)
def _sc_gather(idx_hbm, e_hbm, out_hbm, idx_v, rows_v, packed_v, sem):
    wid = lax.axis_index("s") * _NC + lax.axis_index("c")
    base = wid * _BPW
    pltpu.sync_copy(idx_hbm.at[pl.ds(base, _BPW)], idx_v)
    pltpu.async_copy(e_hbm.at[idx_v], rows_v, sem).wait()

    def _pair(k, carry):
        for half in range(2):
            for j in range(_D // 16):
                v = rows_v[2 * k + half, pl.ds(j * 16, 16)]
                packed_v[k, pl.ds(half * _D + j * 16, 16)] = v
        return carry

    lax.fori_loop(0, _BPW // 2, _pair, 0)
    pltpu.sync_copy(packed_v, out_hbm.at[pl.ds(wid * (_BPW // 2), _BPW // 2)])


def kernel(inputs, embedding_weight):
    enc, idx, loss, ppl = _tc_select(inputs, embedding_weight)
    e_pad = jnp.concatenate(
        [embedding_weight, jnp.zeros((_NE, 128 - _D), jnp.float32)], axis=1)
    qst = _sc_gather(idx.reshape(_N), e_pad).reshape(_N, _D)
    return (loss[0, 0], qst, ppl[0, 0], enc)


# speedup vs baseline: 1.0889x; 1.0889x over previous
import functools

import jax
import jax.numpy as jnp
from jax import lax
from jax.experimental import pallas as pl
from jax.experimental.pallas import tpu as pltpu
from jax.experimental.pallas import tpu_sc as plsc

_NE = 1024
_D = 64
_N = 16384
_CC = 0.25
_BLK = 1024

_NC = 2
_NS = 16
_NW = _NC * _NS
_BPW = _N // _NW


def _tc_body(x_ref, e_ref, enc_ref, idx_ref, loss_ref, ppl_ref,
             counts_ref, sse_ref):
    i = pl.program_id(0)
    x = x_ref[...]
    e = e_ref[...]

    e2 = jnp.sum(e * e, axis=1)
    s = jax.lax.dot_general(x, e, (((1,), (1,)), ((), ())),
                            preferred_element_type=jnp.float32)
    score = 2.0 * s - e2[None, :]
    idx = jnp.argmax(score, axis=1)
    idx_ref[...] = idx[None, None, :]
    onehot = (jax.lax.broadcasted_iota(jnp.int32, (_BLK, _NE), 1)
              == idx[:, None]).astype(jnp.float32)
    enc_ref[...] = onehot

    @pl.when(i == 0)
    def _init():
        counts_ref[...] = jnp.zeros_like(counts_ref)
        sse_ref[0] = 0.0

    counts_ref[...] += jnp.sum(onehot, axis=0, keepdims=True)
    sse_ref[0] += jnp.sum(x * x) - jnp.sum(onehot * score)

    loss_ref[...] = jnp.zeros((1, 1), jnp.float32)
    ppl_ref[...] = jnp.zeros((1, 1), jnp.float32)

    @pl.when(i == pl.num_programs(0) - 1)
    def _final():
        loss_ref[...] = jnp.full((1, 1), _CC * sse_ref[0] / (_N * _D))
        p = counts_ref[...] / _N
        ppl_ref[...] = jnp.exp(-jnp.sum(p * jnp.log(p + 1e-10),
                                        keepdims=True))


def _tc_select(inputs, embedding_weight):
    return pl.pallas_call(
        _tc_body,
        grid=(_N // _BLK,),
        in_specs=[
            pl.BlockSpec((_BLK, _D), lambda i: (i, 0)),
            pl.BlockSpec((_NE, _D), lambda i: (0, 0)),
        ],
        out_specs=[
            pl.BlockSpec((_BLK, _NE), lambda i: (i, 0)),
            pl.BlockSpec((1, 1, _BLK), lambda i: (i, 0, 0)),
            pl.BlockSpec((1, 1), lambda i: (0, 0)),
            pl.BlockSpec((1, 1), lambda i: (0, 0)),
        ],
        out_shape=[
            jax.ShapeDtypeStruct((_N, _NE), jnp.float32),
            jax.ShapeDtypeStruct((_N // _BLK, 1, _BLK), jnp.int32),
            jax.ShapeDtypeStruct((1, 1), jnp.float32),
            jax.ShapeDtypeStruct((1, 1), jnp.float32),
        ],
        scratch_shapes=[
            pltpu.VMEM((1, _NE), jnp.float32),
            pltpu.SMEM((1,), jnp.float32),
        ],
    )(inputs, embedding_weight)


@functools.partial(
    pl.kernel,
    mesh=plsc.VectorSubcoreMesh(core_axis_name="c", subcore_axis_name="s"),
    out_type=jax.ShapeDtypeStruct((_N, _D), jnp.float32),
    scratch_types=[
        pltpu.VMEM((_BPW,), jnp.int32),
        pltpu.VMEM((_BPW, _D), jnp.float32),
        pltpu.SemaphoreType.DMA,
    ],
    compiler_params=pltpu.CompilerParams(use_tc_tiling_on_sc=False),
)
def _sc_gather(idx_hbm, e_hbm, out_hbm, idx_v, rows_v, sem):
    wid = lax.axis_index("s") * _NC + lax.axis_index("c")
    base = wid * _BPW
    pltpu.sync_copy(idx_hbm.at[pl.ds(base, _BPW)], idx_v)
    pltpu.async_copy(e_hbm.at[idx_v], rows_v, sem).wait()
    pltpu.sync_copy(rows_v, out_hbm.at[pl.ds(base, _BPW)])


def kernel(inputs, embedding_weight):
    enc, idx, loss, ppl = _tc_select(inputs, embedding_weight)
    qst = _sc_gather(idx.reshape(_N), embedding_weight)
    return (loss[0, 0], qst, ppl[0, 0], enc)


# speedup vs baseline: 1.7769x; 1.6319x over previous
import jax
import jax.numpy as jnp
from jax.experimental import pallas as pl
from jax.experimental.pallas import tpu as pltpu

_NE = 1024
_D = 64
_N = 16384
_CC = 0.25
_BLK = 1024


def _vq_body(x_ref, e_ref, enc_ref, q_ref, loss_ref, ppl_ref, counts_ref, sse_ref):
    i = pl.program_id(0)
    x = x_ref[...]
    e = e_ref[...]

    e2 = jnp.sum(e * e, axis=1)
    s = jax.lax.dot_general(x, e, (((1,), (1,)), ((), ())),
                            preferred_element_type=jnp.float32)
    score = 2.0 * s - e2[None, :]
    maxv = jnp.max(score, axis=1, keepdims=True)
    onehot = (score >= maxv).astype(jnp.float32)
    enc_ref[...] = onehot

    q = jax.lax.dot_general(onehot, e, (((1,), (0,)), ((), ())),
                            preferred_element_type=jnp.float32)
    d = q - x
    q_ref[...] = x + d

    @pl.when(i == 0)
    def _init():
        counts_ref[...] = jnp.zeros_like(counts_ref)
        sse_ref[0] = 0.0

    counts_ref[...] += jnp.sum(onehot, axis=0, keepdims=True)
    sse_ref[0] += jnp.sum(d * d)

    loss_ref[...] = jnp.zeros((1, 1), jnp.float32)
    ppl_ref[...] = jnp.zeros((1, 1), jnp.float32)

    @pl.when(i == pl.num_programs(0) - 1)
    def _final():
        loss_ref[...] = jnp.full((1, 1), _CC * sse_ref[0] / (_N * _D))
        p = counts_ref[...] / _N
        ppl_ref[...] = jnp.exp(-jnp.sum(p * jnp.log(p + 1e-10),
                                        keepdims=True))


def kernel(inputs, embedding_weight):
    grid = (_N // _BLK,)
    enc, q, loss, ppl = pl.pallas_call(
        _vq_body,
        grid=grid,
        in_specs=[
            pl.BlockSpec((_BLK, _D), lambda i: (i, 0)),
            pl.BlockSpec((_NE, _D), lambda i: (0, 0)),
        ],
        out_specs=[
            pl.BlockSpec((_BLK, _NE), lambda i: (i, 0)),
            pl.BlockSpec((_BLK, _D), lambda i: (i, 0)),
            pl.BlockSpec((1, 1), lambda i: (0, 0)),
            pl.BlockSpec((1, 1), lambda i: (0, 0)),
        ],
        out_shape=[
            jax.ShapeDtypeStruct((_N, _NE), jnp.float32),
            jax.ShapeDtypeStruct((_N, _D), jnp.float32),
            jax.ShapeDtypeStruct((1, 1), jnp.float32),
            jax.ShapeDtypeStruct((1, 1), jnp.float32),
        ],
        scratch_shapes=[
            pltpu.VMEM((1, _NE), jnp.float32),
            pltpu.SMEM((1,), jnp.float32),
        ],
    )(inputs, embedding_weight)
    return (loss[0, 0], q, ppl[0, 0], enc)


# speedup vs baseline: 1.9931x; 1.1217x over previous
import jax
import jax.numpy as jnp
from jax.experimental import pallas as pl
from jax.experimental.pallas import tpu as pltpu

_NE = 1024
_D = 64
_N = 16384
_CC = 0.25
_BLK = 1024
_KA = _D + 8


def _vq_body(x_ref, e_ref, enc_ref, q_ref, loss_ref, ppl_ref,
             counts_ref, sse_ref, ea_ref, xa_ref):
    i = pl.program_id(0)
    x = x_ref[...]
    e = e_ref[...]

    @pl.when(i == 0)
    def _pre():
        counts_ref[...] = jnp.zeros_like(counts_ref)
        sse_ref[0] = 0.0
        e2 = jnp.sum(e * e, axis=1, keepdims=True)
        ea_ref[...] = jnp.concatenate(
            [e, -e2, jnp.zeros((_NE, _KA - _D - 1), jnp.float32)], axis=1)

    xa_ref[...] = jnp.concatenate(
        [x + x, jnp.ones((_BLK, 1), jnp.float32),
         jnp.zeros((_BLK, _KA - _D - 1), jnp.float32)], axis=1)
    score = jax.lax.dot_general(xa_ref[...], ea_ref[...],
                                (((1,), (1,)), ((), ())),
                                preferred_element_type=jnp.float32)
    maxv = jnp.max(score, axis=1, keepdims=True)
    onehot = (score >= maxv).astype(jnp.float32)
    enc_ref[...] = onehot

    q = jax.lax.dot_general(onehot, e, (((1,), (0,)), ((), ())),
                            preferred_element_type=jnp.float32)
    d = q - x
    q_ref[...] = x + d

    counts_ref[...] += jnp.sum(onehot, axis=0, keepdims=True)
    sse_ref[0] += jnp.sum(d * d)

    loss_ref[...] = jnp.zeros((1, 1), jnp.float32)
    ppl_ref[...] = jnp.zeros((1, 1), jnp.float32)

    @pl.when(i == pl.num_programs(0) - 1)
    def _final():
        loss_ref[...] = jnp.full((1, 1), _CC * sse_ref[0] / (_N * _D))
        p = counts_ref[...] / _N
        ppl_ref[...] = jnp.exp(-jnp.sum(p * jnp.log(p + 1e-10),
                                        keepdims=True))


def kernel(inputs, embedding_weight):
    grid = (_N // _BLK,)
    enc, q, loss, ppl = pl.pallas_call(
        _vq_body,
        grid=grid,
        in_specs=[
            pl.BlockSpec((_BLK, _D), lambda i: (i, 0)),
            pl.BlockSpec((_NE, _D), lambda i: (0, 0)),
        ],
        out_specs=[
            pl.BlockSpec((_BLK, _NE), lambda i: (i, 0)),
            pl.BlockSpec((_BLK, _D), lambda i: (i, 0)),
            pl.BlockSpec((1, 1), lambda i: (0, 0)),
            pl.BlockSpec((1, 1), lambda i: (0, 0)),
        ],
        out_shape=[
            jax.ShapeDtypeStruct((_N, _NE), jnp.float32),
            jax.ShapeDtypeStruct((_N, _D), jnp.float32),
            jax.ShapeDtypeStruct((1, 1), jnp.float32),
            jax.ShapeDtypeStruct((1, 1), jnp.float32),
        ],
        scratch_shapes=[
            pltpu.VMEM((1, _NE), jnp.float32),
            pltpu.SMEM((1,), jnp.float32),
            pltpu.VMEM((_NE, _KA), jnp.float32),
            pltpu.VMEM((_BLK, _KA), jnp.float32),
        ],
    )(inputs, embedding_weight)
    return (loss[0, 0], q, ppl[0, 0], enc)


# speedup vs baseline: 1.9983x; 1.0026x over previous
import jax
import jax.numpy as jnp
from jax.experimental import pallas as pl
from jax.experimental.pallas import tpu as pltpu

_NE = 1024
_D = 64
_N = 16384
_CC = 0.25
_BLK = 1024
_KA = _D + 8


def _vq_body(x_ref, e_ref, enc_ref, q_ref, loss_ref, ppl_ref,
             counts_ref, sse_ref, ea_ref, xa_ref):
    i = pl.program_id(0)
    x = x_ref[...]
    e = e_ref[...]

    @pl.when(i == 0)
    def _pre():
        counts_ref[...] = jnp.zeros_like(counts_ref)
        sse_ref[0] = 0.0
        e2 = jnp.sum(e * e, axis=1, keepdims=True)
        hi = jnp.bfloat16(e2).astype(jnp.float32)
        r1 = e2 - hi
        mid = jnp.bfloat16(r1).astype(jnp.float32)
        lo = r1 - mid
        ea_ref[...] = jnp.concatenate(
            [e, -hi, -mid, -lo,
             jnp.zeros((_NE, _KA - _D - 3), jnp.float32)], axis=1)

    xa_ref[...] = jnp.concatenate(
        [x + x, jnp.ones((_BLK, 3), jnp.float32),
         jnp.zeros((_BLK, _KA - _D - 3), jnp.float32)], axis=1)
    score = jax.lax.dot_general(xa_ref[...], ea_ref[...],
                                (((1,), (1,)), ((), ())),
                                preferred_element_type=jnp.float32)
    maxv = jnp.max(score, axis=1, keepdims=True)
    onehot = (score >= maxv).astype(jnp.float32)
    enc_ref[...] = onehot

    q = jax.lax.dot_general(onehot, e, (((1,), (0,)), ((), ())),
                            preferred_element_type=jnp.float32)
    d = q - x
    q_ref[...] = x + d

    counts_ref[...] += jnp.sum(onehot, axis=0, keepdims=True)
    sse_ref[0] += jnp.sum(d * d)

    loss_ref[...] = jnp.zeros((1, 1), jnp.float32)
    ppl_ref[...] = jnp.zeros((1, 1), jnp.float32)

    @pl.when(i == pl.num_programs(0) - 1)
    def _final():
        loss_ref[...] = jnp.full((1, 1), _CC * sse_ref[0] / (_N * _D))
        p = counts_ref[...] / _N
        ppl_ref[...] = jnp.exp(-jnp.sum(p * jnp.log(p + 1e-10),
                                        keepdims=True))


def kernel(inputs, embedding_weight):
    grid = (_N // _BLK,)
    enc, q, loss, ppl = pl.pallas_call(
        _vq_body,
        grid=grid,
        in_specs=[
            pl.BlockSpec((_BLK, _D), lambda i: (i, 0)),
            pl.BlockSpec((_NE, _D), lambda i: (0, 0)),
        ],
        out_specs=[
            pl.BlockSpec((_BLK, _NE), lambda i: (i, 0)),
            pl.BlockSpec((_BLK, _D), lambda i: (i, 0)),
            pl.BlockSpec((1, 1), lambda i: (0, 0)),
            pl.BlockSpec((1, 1), lambda i: (0, 0)),
        ],
        out_shape=[
            jax.ShapeDtypeStruct((_N, _NE), jnp.float32),
            jax.ShapeDtypeStruct((_N, _D), jnp.float32),
            jax.ShapeDtypeStruct((1, 1), jnp.float32),
            jax.ShapeDtypeStruct((1, 1), jnp.float32),
        ],
        scratch_shapes=[
            pltpu.VMEM((1, _NE), jnp.float32),
            pltpu.SMEM((1,), jnp.float32),
            pltpu.VMEM((_NE, _KA), jnp.float32),
            pltpu.VMEM((_BLK, _KA), jnp.float32),
        ],
    )(inputs, embedding_weight)
    return (loss[0, 0], q, ppl[0, 0], enc)


# speedup vs baseline: 2.0802x; 1.0410x over previous
import jax
import jax.numpy as jnp
from jax.experimental import pallas as pl
from jax.experimental.pallas import tpu as pltpu

_NE = 1024
_D = 64
_N = 16384
_CC = 0.25
_BLK = 2048
_KA = _D + 8


def _vq_body(x_ref, e_ref, enc_ref, q_ref, loss_ref, ppl_ref,
             counts_ref, sse_ref, ea_ref, xa_ref):
    i = pl.program_id(0)
    x = x_ref[...]
    e = e_ref[...]

    @pl.when(i == 0)
    def _pre():
        counts_ref[...] = jnp.zeros_like(counts_ref)
        sse_ref[0] = 0.0
        e2 = jnp.sum(e * e, axis=1, keepdims=True)
        hi = jnp.bfloat16(e2).astype(jnp.float32)
        r1 = e2 - hi
        mid = jnp.bfloat16(r1).astype(jnp.float32)
        lo = r1 - mid
        ea_ref[...] = jnp.concatenate(
            [e, -hi, -mid, -lo,
             jnp.zeros((_NE, _KA - _D - 3), jnp.float32)], axis=1)

    xa_ref[...] = jnp.concatenate(
        [x + x, jnp.ones((_BLK, 3), jnp.float32),
         jnp.zeros((_BLK, _KA - _D - 3), jnp.float32)], axis=1)
    score = jax.lax.dot_general(xa_ref[...], ea_ref[...],
                                (((1,), (1,)), ((), ())),
                                preferred_element_type=jnp.float32)
    maxv = jnp.max(score, axis=1, keepdims=True)
    onehot = (score >= maxv).astype(jnp.float32)
    enc_ref[...] = onehot

    q = jax.lax.dot_general(onehot, e, (((1,), (0,)), ((), ())),
                            preferred_element_type=jnp.float32)
    d = q - x
    q_ref[...] = x + d

    counts_ref[...] += jnp.sum(onehot, axis=0, keepdims=True)
    sse_ref[0] += jnp.sum(d * d)

    loss_ref[...] = jnp.zeros((1, 1), jnp.float32)
    ppl_ref[...] = jnp.zeros((1, 1), jnp.float32)

    @pl.when(i == pl.num_programs(0) - 1)
    def _final():
        loss_ref[...] = jnp.full((1, 1), _CC * sse_ref[0] / (_N * _D))
        p = counts_ref[...] / _N
        ppl_ref[...] = jnp.exp(-jnp.sum(p * jnp.log(p + 1e-10),
                                        keepdims=True))


def kernel(inputs, embedding_weight):
    grid = (_N // _BLK,)
    enc, q, loss, ppl = pl.pallas_call(
        _vq_body,
        grid=grid,
        in_specs=[
            pl.BlockSpec((_BLK, _D), lambda i: (i, 0)),
            pl.BlockSpec((_NE, _D), lambda i: (0, 0)),
        ],
        out_specs=[
            pl.BlockSpec((_BLK, _NE), lambda i: (i, 0)),
            pl.BlockSpec((_BLK, _D), lambda i: (i, 0)),
            pl.BlockSpec((1, 1), lambda i: (0, 0)),
            pl.BlockSpec((1, 1), lambda i: (0, 0)),
        ],
        out_shape=[
            jax.ShapeDtypeStruct((_N, _NE), jnp.float32),
            jax.ShapeDtypeStruct((_N, _D), jnp.float32),
            jax.ShapeDtypeStruct((1, 1), jnp.float32),
            jax.ShapeDtypeStruct((1, 1), jnp.float32),
        ],
        scratch_shapes=[
            pltpu.VMEM((1, _NE), jnp.float32),
            pltpu.SMEM((1,), jnp.float32),
            pltpu.VMEM((_NE, _KA), jnp.float32),
            pltpu.VMEM((_BLK, _KA), jnp.float32),
        ],
    )(inputs, embedding_weight)
    return (loss[0, 0], q, ppl[0, 0], enc)


# speedup vs baseline: 2.0901x; 1.0048x over previous
import jax
import jax.numpy as jnp
from jax.experimental import pallas as pl
from jax.experimental.pallas import tpu as pltpu

_NE = 1024
_D = 64
_N = 16384
_CC = 0.25
_BLK = 2048
_KA = _D + 8


def _vq_body(x_ref, e_ref, enc_ref, q_ref, loss_ref, ppl_ref,
             counts_ref, sse_ref, ea_ref):
    i = pl.program_id(0)
    x = x_ref[...]
    e = e_ref[...]

    @pl.when(i == 0)
    def _pre():
        counts_ref[...] = jnp.zeros_like(counts_ref)
        sse_ref[0] = 0.0
        e2 = jnp.sum(e * e, axis=1, keepdims=True)
        hi = jnp.bfloat16(e2).astype(jnp.float32)
        r1 = e2 - hi
        mid = jnp.bfloat16(r1).astype(jnp.float32)
        lo = r1 - mid
        ea_ref[...] = jnp.concatenate(
            [e, -hi, -mid, -lo,
             jnp.zeros((_NE, _KA - _D - 3), jnp.float32)], axis=1)

    xa = jnp.concatenate(
        [x + x, jnp.ones((_BLK, 3), jnp.float32),
         jnp.zeros((_BLK, _KA - _D - 3), jnp.float32)], axis=1)
    score = jax.lax.dot_general(xa, ea_ref[...],
                                (((1,), (1,)), ((), ())),
                                preferred_element_type=jnp.float32)
    maxv = jnp.max(score, axis=1, keepdims=True)
    onehot = (score >= maxv).astype(jnp.float32)
    enc_ref[...] = onehot

    q = jax.lax.dot_general(onehot, e, (((1,), (0,)), ((), ())),
                            preferred_element_type=jnp.float32)
    d = q - x
    q_ref[...] = x + d

    counts_ref[...] += jnp.sum(onehot, axis=0, keepdims=True)
    sse_ref[0] += jnp.sum(d * d)

    loss_ref[...] = jnp.zeros((1, 1), jnp.float32)
    ppl_ref[...] = jnp.zeros((1, 1), jnp.float32)

    @pl.when(i == pl.num_programs(0) - 1)
    def _final():
        loss_ref[...] = jnp.full((1, 1), _CC * sse_ref[0] / (_N * _D))
        p = counts_ref[...] / _N
        ppl_ref[...] = jnp.exp(-jnp.sum(p * jnp.log(p + 1e-10),
                                        keepdims=True))


def kernel(inputs, embedding_weight):
    grid = (_N // _BLK,)
    enc, q, loss, ppl = pl.pallas_call(
        _vq_body,
        grid=grid,
        in_specs=[
            pl.BlockSpec((_BLK, _D), lambda i: (i, 0)),
            pl.BlockSpec((_NE, _D), lambda i: (0, 0)),
        ],
        out_specs=[
            pl.BlockSpec((_BLK, _NE), lambda i: (i, 0)),
            pl.BlockSpec((_BLK, _D), lambda i: (i, 0)),
            pl.BlockSpec((1, 1), lambda i: (0, 0)),
            pl.BlockSpec((1, 1), lambda i: (0, 0)),
        ],
        out_shape=[
            jax.ShapeDtypeStruct((_N, _NE), jnp.float32),
            jax.ShapeDtypeStruct((_N, _D), jnp.float32),
            jax.ShapeDtypeStruct((1, 1), jnp.float32),
            jax.ShapeDtypeStruct((1, 1), jnp.float32),
        ],
        scratch_shapes=[
            pltpu.VMEM((1, _NE), jnp.float32),
            pltpu.SMEM((1,), jnp.float32),
            pltpu.VMEM((_NE, _KA), jnp.float32),
        ],
    )(inputs, embedding_weight)
    return (loss[0, 0], q, ppl[0, 0], enc)
